# baseline (speedup 1.0000x reference)
import functools

import jax
import jax.numpy as jnp
from jax import lax
from jax.experimental import pallas as pl
from jax.experimental.pallas import tpu as pltpu
from jax.experimental.pallas import tpu_sc as plsc

BATCH = 16384
EMBED_DIM = 64
PAIRS = BATCH // 2
PAIR_DIM = 2 * EMBED_DIM

_info = plsc.get_sparse_core_info()
_NW = _info.num_cores * _info.num_subcores
_P_PER_W = PAIRS // _NW
_R_PER_W = BATCH // _NW
_IDX_CHUNK = 128
_LANES = 16


@functools.partial(
    pl.kernel,
    mesh=plsc.VectorSubcoreMesh(core_axis_name="c", subcore_axis_name="s"),
    out_type=jax.ShapeDtypeStruct((PAIRS, PAIR_DIM), jnp.float32),
    scratch_types=[
        pltpu.VMEM((_R_PER_W,), jnp.int32),
        pltpu.VMEM((_P_PER_W,), jnp.int32),
        pltpu.VMEM((_P_PER_W, PAIR_DIM), jnp.float32),
        pltpu.SemaphoreType.DMA,
        pltpu.SemaphoreType.DMA,
    ],
    compiler_params=pltpu.CompilerParams(needs_layout_passes=False),
)
def _pair_lookup(table_hbm, role_hbm, out_hbm, role_v, pair_v, rows_v, gsem, wsem):
    wid = lax.axis_index("s") * _info.num_cores + lax.axis_index("c")
    pltpu.sync_copy(role_hbm.at[pl.ds(wid * _R_PER_W, _R_PER_W)], role_v)

    lane = lax.iota(jnp.int32, _LANES)
    n_chunks = _P_PER_W // _IDX_CHUNK

    gathers = []
    for j in range(n_chunks):
        for k in range(_IDX_CHUNK // _LANES):
            base = j * 2 * _IDX_CHUNK + 2 * _LANES * k
            even = plsc.load_gather(role_v, [base + 2 * lane])
            odd = plsc.load_gather(role_v, [base + 2 * lane + 1])
            pair_v[pl.ds(j * _IDX_CHUNK + _LANES * k, _LANES)] = 2 * even + odd
        gathers.append(
            pltpu.async_copy(
                table_hbm.at[pair_v.at[pl.ds(j * _IDX_CHUNK, _IDX_CHUNK)]],
                rows_v.at[pl.ds(j * _IDX_CHUNK, _IDX_CHUNK)],
                gsem,
            )
        )

    writes = []
    for j in range(n_chunks):
        gathers[j].wait()
        writes.append(
            pltpu.async_copy(
                rows_v.at[pl.ds(j * _IDX_CHUNK, _IDX_CHUNK)],
                out_hbm.at[pl.ds(wid * _P_PER_W + j * _IDX_CHUNK, _IDX_CHUNK)],
                wsem,
            )
        )
    for w in writes:
        w.wait()


def kernel(obs, role, W_role):
    del obs
    table4 = jnp.concatenate(
        [jnp.repeat(W_role, 2, axis=0), jnp.tile(W_role, (2, 1))], axis=1
    )
    out_pairs = _pair_lookup(table4, role)
    return out_pairs.reshape(BATCH, EMBED_DIM)


# speedup vs baseline: 2.3989x; 2.3989x over previous
import functools

import jax
import jax.numpy as jnp
from jax import lax
from jax.experimental import pallas as pl
from jax.experimental.pallas import tpu as pltpu
from jax.experimental.pallas import tpu_sc as plsc

BATCH = 16384
EMBED_DIM = 64
GROUP = 4
QUADS = BATCH // GROUP
QUAD_DIM = GROUP * EMBED_DIM

_info = plsc.get_sparse_core_info()
_NW = _info.num_cores * _info.num_subcores
_Q_PER_W = QUADS // _NW
_R_PER_W = BATCH // _NW
_LANES = 16


@functools.partial(
    pl.kernel,
    mesh=plsc.VectorSubcoreMesh(core_axis_name="c", subcore_axis_name="s"),
    out_type=jax.ShapeDtypeStruct((QUADS, QUAD_DIM), jnp.float32),
    scratch_types=[
        pltpu.VMEM((_R_PER_W,), jnp.int32),
        pltpu.VMEM((_Q_PER_W,), jnp.int32),
        pltpu.VMEM((_Q_PER_W, QUAD_DIM), jnp.float32),
        pltpu.SemaphoreType.DMA,
    ],
    compiler_params=pltpu.CompilerParams(needs_layout_passes=False),
)
def _quad_lookup(table_hbm, role_hbm, out_hbm, role_v, quad_v, rows_v, sem):
    wid = lax.axis_index("s") * _info.num_cores + lax.axis_index("c")
    pltpu.sync_copy(role_hbm.at[pl.ds(wid * _R_PER_W, _R_PER_W)], role_v)

    lane = lax.iota(jnp.int32, _LANES)
    for k in range(_Q_PER_W // _LANES):
        base = GROUP * _LANES * k
        q = plsc.load_gather(role_v, [base + GROUP * lane])
        for d in range(1, GROUP):
            q = 2 * q + plsc.load_gather(role_v, [base + GROUP * lane + d])
        quad_v[pl.ds(_LANES * k, _LANES)] = q

    pltpu.async_copy(table_hbm.at[quad_v], rows_v, sem).wait()
    pltpu.sync_copy(rows_v, out_hbm.at[pl.ds(wid * _Q_PER_W, _Q_PER_W)])


def kernel(obs, role, W_role):
    del obs
    t = W_role
    table16 = jnp.concatenate(
        [
            jnp.repeat(t, 8, axis=0),
            jnp.tile(jnp.repeat(t, 4, axis=0), (2, 1)),
            jnp.tile(jnp.repeat(t, 2, axis=0), (4, 1)),
            jnp.tile(t, (8, 1)),
        ],
        axis=1,
    )
    out_quads = _quad_lookup(table16, role)
    return out_quads.reshape(BATCH, EMBED_DIM)


# speedup vs baseline: 2.8639x; 1.1938x over previous
import functools

import jax
import jax.numpy as jnp
from jax import lax
from jax.experimental import pallas as pl
from jax.experimental.pallas import tpu as pltpu
from jax.experimental.pallas import tpu_sc as plsc

BATCH = 16384
EMBED_DIM = 64
GROUP = 8
QUADS = BATCH // GROUP
QUAD_DIM = GROUP * EMBED_DIM

_info = plsc.get_sparse_core_info()
_NW = _info.num_cores * _info.num_subcores
_Q_PER_W = QUADS // _NW
_R_PER_W = BATCH // _NW
_LANES = 16


@functools.partial(
    pl.kernel,
    mesh=plsc.VectorSubcoreMesh(core_axis_name="c", subcore_axis_name="s"),
    out_type=jax.ShapeDtypeStruct((QUADS, QUAD_DIM), jnp.float32),
    scratch_types=[
        pltpu.VMEM((_R_PER_W,), jnp.int32),
        pltpu.VMEM((_Q_PER_W,), jnp.int32),
        pltpu.VMEM((_Q_PER_W, QUAD_DIM), jnp.float32),
        pltpu.SemaphoreType.DMA,
    ],
    compiler_params=pltpu.CompilerParams(needs_layout_passes=False),
)
def _quad_lookup(table_hbm, role_hbm, out_hbm, role_v, quad_v, rows_v, sem):
    wid = lax.axis_index("s") * _info.num_cores + lax.axis_index("c")
    pltpu.sync_copy(role_hbm.at[pl.ds(wid * _R_PER_W, _R_PER_W)], role_v)

    lane = lax.iota(jnp.int32, _LANES)
    for k in range(_Q_PER_W // _LANES):
        base = GROUP * _LANES * k
        q = plsc.load_gather(role_v, [base + GROUP * lane])
        for d in range(1, GROUP):
            q = 2 * q + plsc.load_gather(role_v, [base + GROUP * lane + d])
        quad_v[pl.ds(_LANES * k, _LANES)] = q

    pltpu.async_copy(table_hbm.at[quad_v], rows_v, sem).wait()
    pltpu.sync_copy(rows_v, out_hbm.at[pl.ds(wid * _Q_PER_W, _Q_PER_W)])


def kernel(obs, role, W_role):
    del obs
    t = W_role
    nrows = 1 << GROUP
    cols = []
    for d in range(GROUP):
        rep = 1 << (GROUP - 1 - d)
        cols.append(jnp.tile(jnp.repeat(t, rep, axis=0), (nrows // (2 * rep), 1)))
    table_g = jnp.concatenate(cols, axis=1)
    out_quads = _quad_lookup(table_g, role)
    return out_quads.reshape(BATCH, EMBED_DIM)
